# Initial kernel scaffold; baseline (speedup 1.0000x reference)
import jax
import jax.numpy as jnp
from jax.experimental import pallas as pl


def kernel(x, params):
    raise NotImplementedError("write your pallas kernel here")


# speedup vs baseline: 13.2143x; 13.2143x over previous
import functools
import math

import jax
import jax.numpy as jnp
from jax import lax
from jax.experimental import pallas as pl
from jax.experimental.pallas import tpu as pltpu
from jax.experimental.pallas import tpu_sc as plsc

B = 32
J3 = 34
T = 256
N_STATE = 32
DEPTH = 3
ROW = J3 * T


def _ln(x, g, b):
    mu = x.mean(-1, keepdims=True)
    var = ((x - mu) ** 2).mean(-1, keepdims=True)
    return (x - mu) / jnp.sqrt(var + 1e-5) * g + b


def _silu(x):
    return x * jax.nn.sigmoid(x)


def _layer_body(x_ref, *refs, L, D, r, final_ln):
    N = N_STATE
    (wixT1, wizT1, cw01, cw11, cb1, wxdT1, wxbT1, wxcT1, wdtT1, bdt1, anT1, dv1, woT1,
     wixT2, wizT2, cw02, cw12, cb2, wxdT2, wxbT2, wxcT2, wdtT2, bdt2, anT2, dv2, woT2,
     ln1g, ln1b, ffw1T, ffb1, ffw2T, ffb2, ln2g, ln2b, fg, fb,
     o_ref, dt_s, xc_s, bm_s, cm_s, ys_s) = refs

    LB = L * B
    x = x_ref[:]
    x2 = x.reshape(LB, D)

    def run_dir(wixT, wizT, cw0, cw1, cb, wxdT, wxbT, wxcT, wdtT, bdt, anT, dv,
                woT, reverse):
        xp = jnp.dot(x2, wixT[:], preferred_element_type=jnp.float32)
        z = jnp.dot(x2, wizT[:], preferred_element_type=jnp.float32)
        xp3 = xp.reshape(L, B, D)
        zero = jnp.zeros((1, B, D), jnp.float32)
        if not reverse:
            xsh = jnp.concatenate([zero, xp3[:-1]], axis=0)
        else:
            xsh = jnp.concatenate([xp3[1:], zero], axis=0)
        xc = xsh * cw0[:] + xp3 * cw1[:] + cb[:]
        xc = _silu(xc)
        xc2 = xc.reshape(LB, D)
        dtl = jnp.dot(xc2, wxdT[:], preferred_element_type=jnp.float32)
        bm = jnp.dot(xc2, wxbT[:], preferred_element_type=jnp.float32)
        cm = jnp.dot(xc2, wxcT[:], preferred_element_type=jnp.float32)
        dtf = jax.nn.softplus(
            jnp.dot(dtl, wdtT[:], preferred_element_type=jnp.float32) + bdt[:])
        dt_s[:] = dtf.reshape(L, B, D)
        xc_s[:] = xc
        bm_s[:] = bm.reshape(L, B, N)
        cm_s[:] = cm.reshape(L, B, N)
        anT_v = anT[:]

        def step(i, h):
            tt = L - 1 - i if reverse else i
            dt_t = dt_s[tt]
            xc_t = xc_s[tt]
            b_t = bm_s[tt]
            c_t = cm_s[tt]
            dA = jnp.exp(dt_t[:, None, :] * anT_v[None, :, :])
            h = dA * h + (dt_t * xc_t)[:, None, :] * b_t[:, :, None]
            ys_s[tt] = jnp.sum(h * c_t[:, :, None], axis=1)
            return h

        lax.fori_loop(0, L, step, jnp.zeros((B, N, D), jnp.float32))
        y = ys_s[:] + xc * dv[:]
        y = y * _silu(z.reshape(L, B, D))
        return jnp.dot(y.reshape(LB, D), woT[:], preferred_element_type=jnp.float32)

    o1 = run_dir(wixT1, wizT1, cw01, cw11, cb1, wxdT1, wxbT1, wxcT1, wdtT1,
                 bdt1, anT1, dv1, woT1, reverse=False)
    o2 = run_dir(wixT2, wizT2, cw02, cw12, cb2, wxdT2, wxbT2, wxcT2, wdtT2,
                 bdt2, anT2, dv2, woT2, reverse=True)

    xr = x + (o1 + o2).reshape(L, B, D)
    x1 = _ln(xr, ln1g[:], ln1b[:])
    ffh = jax.nn.gelu(
        jnp.dot(x1.reshape(LB, D), ffw1T[:], preferred_element_type=jnp.float32)
        + ffb1[:])
    ffo = jnp.dot(ffh, ffw2T[:], preferred_element_type=jnp.float32) + ffb2[:]
    out = _ln(x1 + ffo.reshape(L, B, D), ln2g[:], ln2b[:])
    if final_ln:
        out = _ln(out, fg[:], fb[:])
    o_ref[:] = out


def _prep_mamba(p):
    r = p['W_dt'].shape[1]
    d = p['W_in'].shape[1]
    wiT = p['W_in'].T
    wxT = p['W_x'].T
    return (
        wiT[:, :d], wiT[:, d:],
        p['conv_w'][:, 0], p['conv_w'][:, 1], p['conv_b'],
        wxT[:, :r], wxT[:, r:r + N_STATE], wxT[:, r + N_STATE:],
        p['W_dt'].T, p['b_dt'],
        (-jnp.exp(p['A_log'])).T,
        p['D'], p['W_out'].T,
    )


def _layer_call(x_lbd, lp, fin_g, fin_b, L, D, r, final_ln, interpret=False):
    N = N_STATE
    args = (x_lbd, *_prep_mamba(lp['m1']), *_prep_mamba(lp['m2']),
            lp['ln1_g'], lp['ln1_b'], lp['ff_w1'].T, lp['ff_b1'],
            lp['ff_w2'].T, lp['ff_b2'], lp['ln2_g'], lp['ln2_b'], fin_g, fin_b)
    return pl.pallas_call(
        functools.partial(_layer_body, L=L, D=D, r=r, final_ln=final_ln),
        out_shape=jax.ShapeDtypeStruct((L, B, D), jnp.float32),
        scratch_shapes=[
            pltpu.VMEM((L, B, D), jnp.float32),
            pltpu.VMEM((L, B, D), jnp.float32),
            pltpu.VMEM((L, B, N), jnp.float32),
            pltpu.VMEM((L, B, N), jnp.float32),
            pltpu.VMEM((L, B, D), jnp.float32),
        ],
        interpret=interpret,
    )(*args)


def _encoder(x_lbd, ep, L, D, r, interpret=False):
    h = x_lbd
    for i, lp in enumerate(ep['layers']):
        h = _layer_call(h, lp, ep['ln_g'], ep['ln_b'], L, D, r,
                        final_ln=(i == DEPTH - 1), interpret=interpret)
    return h


def _gate_body(x_ref, gwT_ref, gb_ref, i1_ref, i2_ref):
    xv = x_ref[:]
    pooled = jnp.mean(xv, axis=1)
    logits = jnp.dot(pooled, gwT_ref[:], preferred_element_type=jnp.float32)
    logits = logits + gb_ref[:]
    mx = jnp.max(logits, axis=-1, keepdims=True)
    i4 = lax.broadcasted_iota(jnp.int32, logits.shape, 1)
    e = jnp.min(jnp.where(logits >= mx, i4, 4), axis=-1, keepdims=True)
    first_T = ((e == 0) | (e == 3)).astype(jnp.int32)
    second_S = ((e == 0) | (e == 2)).astype(jnp.int32)
    biota = lax.broadcasted_iota(jnp.int32, (B, 1), 0)
    i1_ref[:] = biota + B * (1 - first_T)
    i2_ref[:] = biota + B * (1 - second_S)


def _gate(x, gw, gb, interpret=False):
    i1, i2 = pl.pallas_call(
        _gate_body,
        out_shape=(jax.ShapeDtypeStruct((B, 1), jnp.int32),
                   jax.ShapeDtypeStruct((B, 1), jnp.int32)),
        interpret=interpret,
    )(x, gw.T, gb)
    return i1.reshape(B), i2.reshape(B)


_B_PER_W = 8
_N_SEL_W = B // _B_PER_W


def _make_select():
    mesh = plsc.VectorSubcoreMesh(core_axis_name="c", subcore_axis_name="s")

    @functools.partial(
        pl.kernel, mesh=mesh,
        out_type=jax.ShapeDtypeStruct((B, ROW), jnp.float32),
        scratch_types=[
            pltpu.VMEM((_B_PER_W,), jnp.int32),
            pltpu.VMEM((_B_PER_W, ROW), jnp.float32),
            pltpu.SemaphoreType.DMA,
        ],
    )
    def sel(table_hbm, idx_hbm, out_hbm, idx_v, rows_v, sem):
        wid = lax.axis_index("s") * 2 + lax.axis_index("c")

        @pl.when(wid < _N_SEL_W)
        def _():
            base = wid * _B_PER_W
            pltpu.sync_copy(idx_hbm.at[pl.ds(base, _B_PER_W)], idx_v)
            pltpu.async_copy(table_hbm.at[idx_v], rows_v, sem).wait()
            pltpu.sync_copy(rows_v, out_hbm.at[pl.ds(base, _B_PER_W)])

    return sel


_SEL_CACHE = []


def _select(table, idx):
    if not _SEL_CACHE:
        _SEL_CACHE.append(_make_select())
    return _SEL_CACHE[0](table, idx)


def kernel(x, params):
    pT = params['enc_T']
    pS = params['enc_S']
    rT = pT['layers'][0]['m1']['W_dt'].shape[1]
    rS = pS['layers'][0]['m1']['W_dt'].shape[1]

    u = _encoder(x.transpose(2, 0, 1), pT, L=T, D=J3, r=rT)
    v = _encoder(x.transpose(1, 0, 2), pS, L=J3, D=T, r=rS)

    i1, i2 = _gate(x, params['gate_w'], params['gate_b'])

    u_rows = u.transpose(1, 2, 0).reshape(B, ROW)
    v_rows = v.transpose(1, 0, 2).reshape(B, ROW)
    y1 = _select(jnp.concatenate([u_rows, v_rows], axis=0), i1)
    y1 = y1.reshape(B, J3, T)

    a = _encoder(y1.transpose(1, 0, 2), pS, L=J3, D=T, r=rS)
    c = _encoder(y1.transpose(2, 0, 1), pT, L=T, D=J3, r=rT)
    a_rows = a.transpose(1, 0, 2).reshape(B, ROW)
    c_rows = c.transpose(1, 2, 0).reshape(B, ROW)
    out = _select(jnp.concatenate([a_rows, c_rows], axis=0), i2)
    return out.reshape(B, J3, T)
